# Initial kernel scaffold; baseline (speedup 1.0000x reference)
import jax
import jax.numpy as jnp
from jax.experimental import pallas as pl


def kernel(x, codebook, proj_kernel, proj_bias):
    raise NotImplementedError("write your pallas kernel here")


# speedup vs baseline: 1.4377x; 1.4377x over previous
import functools

import jax
import jax.numpy as jnp
from jax import lax
from jax.experimental import pallas as pl
from jax.experimental.pallas import tpu as pltpu
from jax.experimental.pallas import tpu_sc as plsc

_NUM_CODES = 8192
_HALF = _NUM_CODES // 2
_DIM = 64
_TM = 256


def _vq_tile_kernel(x_ref, cb_ref, w_ref, b_ref, disc_ref, idx_ref):
    wb = w_ref[...]
    b = b_ref[...]
    xp = jnp.dot(x_ref[...], wb, preferred_element_type=jnp.float32) + b
    cbp = jnp.dot(cb_ref[...], wb, preferred_element_type=jnp.float32) + b
    xn = jnp.sum(xp * xp, axis=1, keepdims=True)
    cn = jnp.sum(cbp * cbp, axis=1)
    mm = lax.dot_general(xp.astype(jnp.bfloat16), cbp.astype(jnp.bfloat16),
                         (((1,), (1,)), ((), ())),
                         preferred_element_type=jnp.float32)
    dist = (xn + (-2.0) * mm) + cn[None, :]

    d0 = dist[:, :_HALF]
    d1 = dist[:, _HALF:]
    m0 = jnp.min(d0, axis=1)
    m1 = jnp.min(d1, axis=1)
    hiota = lax.broadcasted_iota(jnp.int32, (_TM, _HALF), 1)
    i0 = jnp.min(jnp.where(d0 == m0[:, None], hiota, _HALF), axis=1)
    i1 = jnp.min(jnp.where(d1 == m1[:, None], hiota, _HALF), axis=1) + _HALF
    m0u = lax.bitcast_convert_type(m0, jnp.uint32) + jnp.uint32(0x7FFF)
    m0r = lax.bitcast_convert_type(m0u & jnp.uint32(0xFFFF0000), jnp.float32)
    take1 = m1 < m0r
    idx = jnp.where(take1, i1, i0)

    iota = lax.broadcasted_iota(jnp.int32, (_TM, _NUM_CODES), 1)
    disc_ref[...] = (iota == idx[:, None]).astype(jnp.float32)
    idx_ref[...] = idx.reshape(1, 1, _TM)


_GW = 128
_CHUNK = 96


def _make_sc_gather(num_rows):
    info = plsc.get_sparse_core_info()
    nw = info.num_cores * info.num_subcores
    rows_per_w = num_rows // nw
    n_chunks = rows_per_w // _CHUNK
    mesh = plsc.VectorSubcoreMesh(core_axis_name="c", subcore_axis_name="s")

    @functools.partial(
        pl.kernel,
        mesh=mesh,
        out_type=jax.ShapeDtypeStruct((num_rows, _GW), jnp.float32),
        scratch_types=[
            pltpu.VMEM((rows_per_w,), jnp.int32),
            pltpu.VMEM((rows_per_w, _GW), jnp.float32),
            pltpu.SemaphoreType.DMA,
        ],
    )
    def gather_kernel(cb_hbm, idx_hbm, out_hbm, idx_v, rows_v, sem):
        wid = lax.axis_index("s") * info.num_cores + lax.axis_index("c")
        base = wid * rows_per_w
        pltpu.sync_copy(idx_hbm.at[pl.ds(base, rows_per_w)], idx_v)
        copies = [
            pltpu.async_copy(
                cb_hbm.at[idx_v.at[pl.ds(j * _CHUNK, _CHUNK)]],
                rows_v.at[pl.ds(j * _CHUNK, _CHUNK)],
                sem,
            )
            for j in range(n_chunks)
        ]
        for c in copies:
            c.wait()
        pltpu.sync_copy(rows_v, out_hbm.at[pl.ds(base, rows_per_w)])

    return gather_kernel


def kernel(x, codebook, proj_kernel, proj_bias):
    n = x.shape[0] * x.shape[1]
    xb = x.reshape(n, _DIM).astype(jnp.bfloat16)
    cbb = codebook.astype(jnp.bfloat16)
    wb = proj_kernel.astype(jnp.bfloat16)
    nt = n // _TM
    disc, idx3 = pl.pallas_call(
        _vq_tile_kernel,
        grid=(nt,),
        in_specs=[
            pl.BlockSpec((_TM, _DIM), lambda i: (i, 0)),
            pl.BlockSpec((_NUM_CODES, _DIM), lambda i: (0, 0)),
            pl.BlockSpec((_DIM, _DIM), lambda i: (0, 0)),
            pl.BlockSpec((1, _DIM), lambda i: (0, 0)),
        ],
        out_specs=[
            pl.BlockSpec((_TM, _NUM_CODES), lambda i: (i, 0)),
            pl.BlockSpec((1, 1, _TM), lambda i: (i, 0, 0)),
        ],
        out_shape=[
            jax.ShapeDtypeStruct((n, _NUM_CODES), jnp.float32),
            jax.ShapeDtypeStruct((nt, 1, _TM), jnp.int32),
        ],
        compiler_params=pltpu.CompilerParams(
            dimension_semantics=("arbitrary",),
        ),
    )(xb, cbb, wb, proj_bias.reshape(1, _DIM))
    idx_flat = idx3.reshape(n)
    cb_rounded = codebook.astype(jnp.bfloat16).astype(jnp.float32)
    cb_pad = jnp.pad(cb_rounded, ((0, 0), (0, _GW - _DIM)))
    quant = _make_sc_gather(n)(cb_pad, idx_flat)
    return disc, quant[:, :_DIM].reshape(x.shape)


# speedup vs baseline: 1.7922x; 1.2466x over previous
import functools

import jax
import jax.numpy as jnp
from jax import lax
from jax.experimental import pallas as pl
from jax.experimental.pallas import tpu as pltpu
from jax.experimental.pallas import tpu_sc as plsc

_NUM_CODES = 8192
_HALF = _NUM_CODES // 2
_DIM = 64
_TM = 512


def _codebook_proj_kernel(cbt_ref, wt_ref, b_ref, cbpt_ref, cnt_ref):
    cbpt = jnp.dot(wt_ref[...], cbt_ref[...],
                   preferred_element_type=jnp.float32) + b_ref[...]
    cnt_ref[...] = jnp.sum(cbpt * cbpt, axis=0, keepdims=True)
    cbpt_ref[...] = cbpt.astype(jnp.bfloat16)


def _vq_tile_kernel(x_ref, w_ref, b_ref, cbpt_ref, cnt_ref, disc_ref, idx_ref):
    xp = jnp.dot(x_ref[...], w_ref[...],
                 preferred_element_type=jnp.float32) + b_ref[...]
    xn = jnp.sum(xp * xp, axis=1, keepdims=True)
    mm = jnp.dot(xp.astype(jnp.bfloat16), cbpt_ref[...],
                 preferred_element_type=jnp.float32)
    dist = (xn + (-2.0) * mm) + cnt_ref[...]

    d0 = dist[:, :_HALF]
    d1 = dist[:, _HALF:]
    m0 = jnp.min(d0, axis=1)
    m1 = jnp.min(d1, axis=1)
    hiota = lax.broadcasted_iota(jnp.int32, (_TM, _HALF), 1)
    i0 = jnp.min(jnp.where(d0 == m0[:, None], hiota, _HALF), axis=1)
    i1 = jnp.min(jnp.where(d1 == m1[:, None], hiota, _HALF), axis=1) + _HALF
    m0u = lax.bitcast_convert_type(m0, jnp.uint32) + jnp.uint32(0x7FFF)
    m0r = lax.bitcast_convert_type(m0u & jnp.uint32(0xFFFF0000), jnp.float32)
    take1 = m1 < m0r
    idx = jnp.where(take1, i1, i0)

    iota = lax.broadcasted_iota(jnp.int32, (_TM, _NUM_CODES), 1)
    disc_ref[...] = (iota == idx[:, None]).astype(jnp.float32)
    idx_ref[...] = idx.reshape(1, 1, _TM)


_GW = 128
_CHUNK = 96


def _make_sc_gather(num_rows):
    info = plsc.get_sparse_core_info()
    nw = info.num_cores * info.num_subcores
    rows_per_w = num_rows // nw
    n_chunks = rows_per_w // _CHUNK
    mesh = plsc.VectorSubcoreMesh(core_axis_name="c", subcore_axis_name="s")

    @functools.partial(
        pl.kernel,
        mesh=mesh,
        out_type=jax.ShapeDtypeStruct((num_rows, _GW), jnp.float32),
        scratch_types=[
            pltpu.VMEM((rows_per_w,), jnp.int32),
            pltpu.VMEM((rows_per_w, _GW), jnp.float32),
            pltpu.SemaphoreType.DMA,
        ],
    )
    def gather_kernel(cb_hbm, idx_hbm, out_hbm, idx_v, rows_v, sem):
        wid = lax.axis_index("s") * info.num_cores + lax.axis_index("c")
        base = wid * rows_per_w
        pltpu.sync_copy(idx_hbm.at[pl.ds(base, rows_per_w)], idx_v)
        copies = [
            pltpu.async_copy(
                cb_hbm.at[idx_v.at[pl.ds(j * _CHUNK, _CHUNK)]],
                rows_v.at[pl.ds(j * _CHUNK, _CHUNK)],
                sem,
            )
            for j in range(n_chunks)
        ]
        for c in copies:
            c.wait()
        pltpu.sync_copy(rows_v, out_hbm.at[pl.ds(base, rows_per_w)])

    return gather_kernel


def kernel(x, codebook, proj_kernel, proj_bias):
    n = x.shape[0] * x.shape[1]
    xb = x.reshape(n, _DIM).astype(jnp.bfloat16)
    wb = proj_kernel.astype(jnp.bfloat16)
    cbt = codebook.astype(jnp.bfloat16).T
    b_row = proj_bias.reshape(1, _DIM)
    b_col = proj_bias.reshape(_DIM, 1)

    cbpt_b, cnt = pl.pallas_call(
        _codebook_proj_kernel,
        in_specs=[
            pl.BlockSpec((_DIM, _NUM_CODES), lambda: (0, 0)),
            pl.BlockSpec((_DIM, _DIM), lambda: (0, 0)),
            pl.BlockSpec((_DIM, 1), lambda: (0, 0)),
        ],
        out_specs=[
            pl.BlockSpec((_DIM, _NUM_CODES), lambda: (0, 0)),
            pl.BlockSpec((1, _NUM_CODES), lambda: (0, 0)),
        ],
        out_shape=[
            jax.ShapeDtypeStruct((_DIM, _NUM_CODES), jnp.bfloat16),
            jax.ShapeDtypeStruct((1, _NUM_CODES), jnp.float32),
        ],
    )(cbt, wb.T, b_col)

    nt = n // _TM
    disc, idx3 = pl.pallas_call(
        _vq_tile_kernel,
        grid=(nt,),
        in_specs=[
            pl.BlockSpec((_TM, _DIM), lambda i: (i, 0)),
            pl.BlockSpec((_DIM, _DIM), lambda i: (0, 0)),
            pl.BlockSpec((1, _DIM), lambda i: (0, 0)),
            pl.BlockSpec((_DIM, _NUM_CODES), lambda i: (0, 0)),
            pl.BlockSpec((1, _NUM_CODES), lambda i: (0, 0)),
        ],
        out_specs=[
            pl.BlockSpec((_TM, _NUM_CODES), lambda i: (i, 0)),
            pl.BlockSpec((1, 1, _TM), lambda i: (i, 0, 0)),
        ],
        out_shape=[
            jax.ShapeDtypeStruct((n, _NUM_CODES), jnp.float32),
            jax.ShapeDtypeStruct((nt, 1, _TM), jnp.int32),
        ],
        compiler_params=pltpu.CompilerParams(
            dimension_semantics=("arbitrary",),
        ),
    )(xb, wb, b_row, cbpt_b, cnt)

    idx_flat = idx3.reshape(n)
    cb_rounded = codebook.astype(jnp.bfloat16).astype(jnp.float32)
    cb_pad = jnp.pad(cb_rounded, ((0, 0), (0, _GW - _DIM)))
    quant = _make_sc_gather(n)(cb_pad, idx_flat)
    return disc, quant[:, :_DIM].reshape(x.shape)
